# baseline (speedup 1.0000x reference)
import functools

import jax
import jax.numpy as jnp
from jax import lax
from jax.experimental import pallas as pl
from jax.experimental.pallas import tpu as pltpu
from jax.experimental.pallas import tpu_sc as plsc

_N = 1048576
_CAMS = 200
_CAMS_PAD = 208
_NC, _NS, _L = 2, 16, 16
_NW = _NC * _NS
_CH = 4096
_VPC = _CH // _L

_f32 = jnp.float32
_i32 = jnp.int32

_TS = 11


def _make_sc_body(n):
    rpw = n // _NW
    nch = rpw // _CH

    def _sc_body(win, c2w, kmat,
                 o4h, v4h, nh, pah, ch,
                 inb0, inb1, ob0, ob1, vb0, vb1, nb0, nb1, cb0, cb1,
                 ones_v, tab_v, c2w_v, k_v,
                 si0, si1, so0, so1):
        cid = lax.axis_index("c")
        sid = lax.axis_index("s")
        wid = sid * _NC + cid
        base0 = wid * rpw
        iota = lax.iota(_i32, _L)

        pltpu.sync_copy(c2w, c2w_v)
        pltpu.sync_copy(kmat, k_v)

        def fill_ones(i, carry):
            ones_v[pl.ds(i * _L, _L)] = jnp.full((_L,), 1.0, _f32)
            return carry

        lax.fori_loop(0, _CH // _L, fill_ones, 0)

        def prep(vi, carry):
            cams = vi * _L + iota
            camc = jnp.minimum(cams, _CAMS - 1)
            k9 = camc * 9
            c16 = camc * 16
            tt = cams * _TS

            def gk(col):
                return plsc.load_gather(k_v, [k9 + col])

            def gc(col):
                return plsc.load_gather(c2w_v, [c16 + col])

            fx = gk(0)
            cx = gk(2)
            fy = gk(4)
            cy = gk(5)
            ax = 0.5 - cx
            ay = 0.5 - cy
            for j in range(3):
                aj = gc(4 * j) / fx
                bj = gc(4 * j + 1) / fy
                cj = gc(4 * j + 2) + aj * ax + bj * ay
                tj = gc(4 * j + 3)
                ai = lax.bitcast_convert_type(aj, _i32)
                bi = lax.bitcast_convert_type(bj, _i32)
                ab = ((bi + 0x8000) & (-65536)) | lax.shift_right_logical(
                    ai + 0x8000, 16)
                plsc.store_scatter(tab_v, [tt + j],
                                   lax.bitcast_convert_type(ab, _f32))
                plsc.store_scatter(tab_v, [tt + (3 + j)], cj)
                plsc.store_scatter(tab_v, [tt + (6 + j)], tj)
            return carry

        with jax.named_scope("prep"):
            lax.fori_loop(0, _CAMS_PAD // _L, prep, 0)

        inbufs = (inb0, inb1)
        obufs = (ob0, ob1)
        vbufs = (vb0, vb1)
        nbufs = (nb0, nb1)
        cbufs = (cb0, cb1)
        isems = (si0, si1)
        osems = (so0, so1)

        def start_in(k):
            b = k & 1
            base = base0 + k * _CH
            return [pltpu.async_copy(win.at[pl.ds(4 * base, 4 * _CH)],
                                     inbufs[b], isems[b])]

        def start_out(k):
            b = k & 1
            base = base0 + k * _CH
            return [
                pltpu.async_copy(obufs[b],
                                 o4h.at[pl.ds(4 * base, 4 * _CH)], osems[b]),
                pltpu.async_copy(vbufs[b],
                                 v4h.at[pl.ds(4 * base, 4 * _CH)], osems[b]),
                pltpu.async_copy(nbufs[b],
                                 nh.at[pl.ds(base, _CH)], osems[b]),
                pltpu.async_copy(ones_v, pah.at[pl.ds(base, _CH)], osems[b]),
                pltpu.async_copy(cbufs[b],
                                 ch.at[pl.ds(base, _CH)], osems[b]),
            ]

        in_d = {0: start_in(0)}
        out_d = {}
        for k in range(nch):
            if k + 1 < nch:
                in_d[k + 1] = start_in(k + 1)
            with jax.named_scope(f"wait_in{k}"):
                for d in in_d.pop(k):
                    d.wait()
                if k - 2 in out_d:
                    for d in out_d.pop(k - 2):
                        d.wait()
            inb = inbufs[k & 1]
            ob = obufs[k & 1]
            vb = vbufs[k & 1]
            nb = nbufs[k & 1]
            cb = cbufs[k & 1]

            ns = jax.named_scope(f"vec{k}")
            ns.__enter__()

            @plsc.parallel_loop(0, _VPC, unroll=4)
            def vec(i):
                tpi = (lax.shift_right_logical(i, 3) * 512
                       + (i & 7) * _L)
                c = inb[pl.ds(tpi, _L)]
                y = inb[pl.ds(tpi + 128, _L)].astype(_f32)
                x = inb[pl.ds(tpi + 256, _L)].astype(_f32)
                ct = c * _TS

                def gt(col):
                    return plsc.load_gather(tab_v, [ct + col])

                def gab(col):
                    ab = lax.bitcast_convert_type(gt(col), _i32)
                    a = lax.bitcast_convert_type(
                        lax.shift_left(ab, 16), _f32)
                    b = lax.bitcast_convert_type(ab & (-65536), _f32)
                    return a, b

                a0, b0 = gab(0)
                a1, b1 = gab(1)
                a2, b2 = gab(2)
                d0 = a0 * x + b0 * y + gt(3)
                d1 = a1 * x + b1 * y + gt(4)
                d2 = a2 * x + b2 * y + gt(5)
                s = d0 * d0 + d1 * d1 + d2 * d2 + 1e-30
                bi = lax.bitcast_convert_type(s, _i32)
                r = lax.bitcast_convert_type(
                    0x5F3759DF - lax.shift_right_logical(bi, 1), _f32)
                hs = 0.5 * s
                for _ in range(2):
                    r = r * (1.5 - hs * r * r)
                nrm = s * r
                inv = 1.0 / (nrm + 1e-8)
                tp = tpi
                ob[pl.ds(tp, _L)] = gt(6)
                ob[pl.ds(tp + 128, _L)] = gt(7)
                ob[pl.ds(tp + 256, _L)] = gt(8)
                vb[pl.ds(tp, _L)] = d0 * inv
                vb[pl.ds(tp + 128, _L)] = d1 * inv
                vb[pl.ds(tp + 256, _L)] = d2 * inv
                nb[pl.ds(i * _L, _L)] = nrm
                cb[pl.ds(i * _L, _L)] = c

            ns.__exit__(None, None, None)
            out_d[k] = start_out(k)

        for k in sorted(out_d):
            for d in out_d.pop(k):
                d.wait()

    return _sc_body


def _make_sc_call(n):
    mesh = plsc.VectorSubcoreMesh(core_axis_name="c", subcore_axis_name="s")
    return functools.partial(
        pl.kernel,
        mesh=mesh,
        compiler_params=pltpu.CompilerParams(needs_layout_passes=False),
        out_type=[
            jax.ShapeDtypeStruct((4 * n,), _f32),
            jax.ShapeDtypeStruct((4 * n,), _f32),
            jax.ShapeDtypeStruct((n,), _f32),
            jax.ShapeDtypeStruct((n,), _f32),
            jax.ShapeDtypeStruct((n,), _i32),
        ],
        scratch_types=[
            pltpu.VMEM((4 * _CH,), _i32),
            pltpu.VMEM((4 * _CH,), _i32),
            pltpu.VMEM((4 * _CH,), _f32),
            pltpu.VMEM((4 * _CH,), _f32),
            pltpu.VMEM((4 * _CH,), _f32),
            pltpu.VMEM((4 * _CH,), _f32),
            pltpu.VMEM((_CH,), _f32),
            pltpu.VMEM((_CH,), _f32),
            pltpu.VMEM((_CH,), _i32),
            pltpu.VMEM((_CH,), _i32),
            pltpu.VMEM((_CH,), _f32),
            pltpu.VMEM((_CAMS_PAD * _TS,), _f32),
            pltpu.VMEM((_CAMS * 16,), _f32),
            pltpu.VMEM((_CAMS * 9,), _f32),
            pltpu.SemaphoreType.DMA,
            pltpu.SemaphoreType.DMA,
            pltpu.SemaphoreType.DMA,
            pltpu.SemaphoreType.DMA,
        ],
    )(_make_sc_body(n))


@jax.jit
def _sc_call(win, c2w1, k1):
    return _make_sc_call(_N)(win, c2w1, k1)


def kernel(ray_indices, camera_to_worlds, intrinsics):
    r4 = jnp.pad(ray_indices, ((0, 0), (0, 1)))
    win = r4.reshape(_N // 128, 128, 4).transpose(0, 2, 1).reshape(-1)
    c2w1 = camera_to_worlds.reshape(-1)
    k1 = intrinsics.reshape(-1)
    o4, v4, n1, pa, cin = _sc_call(win, c2w1, k1)

    def untile(x):
        return (x.reshape(_N // 128, 4, 128).transpose(0, 2, 1)
                .reshape(_N, 4)[:, :3])

    origins = untile(o4)
    viewdirs = untile(v4)
    dnorm = n1.reshape(_N, 1)
    pixel_area = pa.reshape(_N, 1)
    return origins, viewdirs, dnorm, pixel_area, cin


# speedup vs baseline: 1.1601x; 1.1601x over previous
import functools

import jax
import jax.numpy as jnp
from jax import lax
from jax.experimental import pallas as pl
from jax.experimental.pallas import tpu as pltpu
from jax.experimental.pallas import tpu_sc as plsc

_N = 1048576
_CAMS = 200
_CAMS_PAD = 208
_NC, _NS, _L = 2, 16, 16
_NW = _NC * _NS
_CH = 4096
_VPC = _CH // _L

_f32 = jnp.float32
_i32 = jnp.int32

_TS = 17


def _make_sc_body(n):
    rpw = n // _NW
    nch = rpw // _CH

    def _sc_body(win, c2w, kmat,
                 o4h, v4h, nh, pah, ch,
                 inb0, inb1, ob0, ob1, vb0, vb1, nb0, nb1, cb0, cb1,
                 ones_v, tab_v, c2w_v, k_v,
                 si0, si1, so0, so1):
        cid = lax.axis_index("c")
        sid = lax.axis_index("s")
        wid = sid * _NC + cid
        base0 = wid * rpw
        iota = lax.iota(_i32, _L)

        pltpu.sync_copy(c2w, c2w_v)
        pltpu.sync_copy(kmat, k_v)

        def fill_ones(i, carry):
            ones_v[pl.ds(i * _L, _L)] = jnp.full((_L,), 1.0, _f32)
            return carry

        lax.fori_loop(0, _CH // _L, fill_ones, 0)

        def prep(vi, carry):
            cams = vi * _L + iota
            camc = jnp.minimum(cams, _CAMS - 1)
            k9 = camc * 9
            c16 = camc * 16
            tt = cams * _TS

            def gk(col):
                return plsc.load_gather(k_v, [k9 + col])

            def gc(col):
                return plsc.load_gather(c2w_v, [c16 + col])

            fx = gk(0)
            cx = gk(2)
            fy = gk(4)
            cy = gk(5)
            ax = 0.5 - cx
            ay = 0.5 - cy
            for j in range(3):
                aj = gc(4 * j) / fx
                bj = gc(4 * j + 1) / fy
                cj = gc(4 * j + 2) + aj * ax + bj * ay
                tj = gc(4 * j + 3)
                plsc.store_scatter(tab_v, [tt + j], aj)
                plsc.store_scatter(tab_v, [tt + (3 + j)], bj)
                plsc.store_scatter(tab_v, [tt + (6 + j)], cj)
                plsc.store_scatter(tab_v, [tt + (9 + j)], tj)
            return carry

        with jax.named_scope("prep"):
            lax.fori_loop(0, _CAMS_PAD // _L, prep, 0)

        inbufs = (inb0, inb1)
        obufs = (ob0, ob1)
        vbufs = (vb0, vb1)
        nbufs = (nb0, nb1)
        cbufs = (cb0, cb1)
        isems = (si0, si1)
        osems = (so0, so1)

        def start_in(k):
            b = k & 1
            base = base0 + k * _CH
            return [pltpu.async_copy(win.at[pl.ds(4 * base, 4 * _CH)],
                                     inbufs[b], isems[b])]

        def start_out(k):
            b = k & 1
            base = base0 + k * _CH
            return [
                pltpu.async_copy(obufs[b],
                                 o4h.at[pl.ds(4 * base, 4 * _CH)], osems[b]),
                pltpu.async_copy(vbufs[b],
                                 v4h.at[pl.ds(4 * base, 4 * _CH)], osems[b]),
                pltpu.async_copy(nbufs[b],
                                 nh.at[pl.ds(base, _CH)], osems[b]),
                pltpu.async_copy(ones_v, pah.at[pl.ds(base, _CH)], osems[b]),
                pltpu.async_copy(cbufs[b],
                                 ch.at[pl.ds(base, _CH)], osems[b]),
            ]

        in_d = {0: start_in(0)}
        out_d = {}
        for k in range(nch):
            if k + 1 < nch:
                in_d[k + 1] = start_in(k + 1)
            with jax.named_scope(f"wait_in{k}"):
                for d in in_d.pop(k):
                    d.wait()
                if k - 2 in out_d:
                    for d in out_d.pop(k - 2):
                        d.wait()
            inb = inbufs[k & 1]
            ob = obufs[k & 1]
            vb = vbufs[k & 1]
            nb = nbufs[k & 1]
            cb = cbufs[k & 1]

            ns = jax.named_scope(f"vec{k}")
            ns.__enter__()

            @plsc.parallel_loop(0, _VPC, unroll=4)
            def vec(i):
                tpi = (lax.shift_right_logical(i, 3) * 512
                       + (i & 7) * _L)
                c = inb[pl.ds(tpi, _L)]
                y = inb[pl.ds(tpi + 128, _L)].astype(_f32)
                x = inb[pl.ds(tpi + 256, _L)].astype(_f32)
                ct = c * _TS

                def gt(col):
                    return plsc.load_gather(tab_v, [ct + col])

                d0 = gt(0) * x + gt(3) * y + gt(6)
                d1 = gt(1) * x + gt(4) * y + gt(7)
                d2 = gt(2) * x + gt(5) * y + gt(8)
                s = d0 * d0 + d1 * d1 + d2 * d2 + 1e-30
                bi = lax.bitcast_convert_type(s, _i32)
                r = lax.bitcast_convert_type(
                    0x5F3759DF - lax.shift_right_logical(bi, 1), _f32)
                hs = 0.5 * s
                for _ in range(2):
                    r = r * (1.5 - hs * r * r)
                nrm = s * r
                inv = 1.0 / (nrm + 1e-8)
                tp = tpi
                ob[pl.ds(tp, _L)] = gt(9)
                ob[pl.ds(tp + 128, _L)] = gt(10)
                ob[pl.ds(tp + 256, _L)] = gt(11)
                vb[pl.ds(tp, _L)] = d0 * inv
                vb[pl.ds(tp + 128, _L)] = d1 * inv
                vb[pl.ds(tp + 256, _L)] = d2 * inv
                nb[pl.ds(i * _L, _L)] = nrm
                cb[pl.ds(i * _L, _L)] = c

            ns.__exit__(None, None, None)
            out_d[k] = start_out(k)

        for k in sorted(out_d):
            for d in out_d.pop(k):
                d.wait()

    return _sc_body


def _make_sc_call(n):
    mesh = plsc.VectorSubcoreMesh(core_axis_name="c", subcore_axis_name="s")
    return functools.partial(
        pl.kernel,
        mesh=mesh,
        compiler_params=pltpu.CompilerParams(needs_layout_passes=False),
        out_type=[
            jax.ShapeDtypeStruct((4 * n,), _f32),
            jax.ShapeDtypeStruct((4 * n,), _f32),
            jax.ShapeDtypeStruct((n,), _f32),
            jax.ShapeDtypeStruct((n,), _f32),
            jax.ShapeDtypeStruct((n,), _i32),
        ],
        scratch_types=[
            pltpu.VMEM((4 * _CH,), _i32),
            pltpu.VMEM((4 * _CH,), _i32),
            pltpu.VMEM((4 * _CH,), _f32),
            pltpu.VMEM((4 * _CH,), _f32),
            pltpu.VMEM((4 * _CH,), _f32),
            pltpu.VMEM((4 * _CH,), _f32),
            pltpu.VMEM((_CH,), _f32),
            pltpu.VMEM((_CH,), _f32),
            pltpu.VMEM((_CH,), _i32),
            pltpu.VMEM((_CH,), _i32),
            pltpu.VMEM((_CH,), _f32),
            pltpu.VMEM((_CAMS_PAD * _TS,), _f32),
            pltpu.VMEM((_CAMS * 16,), _f32),
            pltpu.VMEM((_CAMS * 9,), _f32),
            pltpu.SemaphoreType.DMA,
            pltpu.SemaphoreType.DMA,
            pltpu.SemaphoreType.DMA,
            pltpu.SemaphoreType.DMA,
        ],
    )(_make_sc_body(n))


@jax.jit
def _sc_call(win, c2w1, k1):
    return _make_sc_call(_N)(win, c2w1, k1)


def kernel(ray_indices, camera_to_worlds, intrinsics):
    r4 = jnp.pad(ray_indices, ((0, 0), (0, 1)))
    win = r4.reshape(_N // 128, 128, 4).transpose(0, 2, 1).reshape(-1)
    c2w1 = camera_to_worlds.reshape(-1)
    k1 = intrinsics.reshape(-1)
    o4, v4, n1, pa, cin = _sc_call(win, c2w1, k1)

    def untile(x):
        return (x.reshape(_N // 128, 4, 128).transpose(0, 2, 1)
                .reshape(_N, 4)[:, :3])

    origins = untile(o4)
    viewdirs = untile(v4)
    dnorm = n1.reshape(_N, 1)
    pixel_area = pa.reshape(_N, 1)
    return origins, viewdirs, dnorm, pixel_area, cin
